# baseline (speedup 1.0000x reference)
import functools
import jax
import jax.numpy as jnp
from jax import lax
from jax.experimental import pallas as pl
from jax.experimental.pallas import tpu as pltpu
from jax.experimental.pallas import tpu_sc as plsc

EPS = 1e-06
NUM_GROUPS = 4
D = 1024
GRP = 16


def _rsqrt16(v):
    i = plsc.bitcast(v, jnp.int32)
    i = jnp.int32(0x5F3759DF) - (i >> 1)
    y = plsc.bitcast(i, jnp.float32)
    half = v * 0.5
    for _ in range(3):
        y = y * (1.5 - half * y * y)
    return y


def _sc_body(x_hbm, tt_hbm, gb_hbm, o_hbm, in_v, tt_v, gb_v, out_v):
    n_elts = x_hbm.shape[0]
    n_tok = n_elts // D
    nw = 32
    rows_per_w = n_tok // nw
    wid = lax.axis_index("s") * 2 + lax.axis_index("c")
    base = wid * rows_per_w
    pltpu.sync_copy(gb_hbm, gb_v)
    row_off = lax.iota(jnp.int32, 16) * D

    def group_body(g, _):
        row0 = base + g * GRP
        pltpu.sync_copy(x_hbm.at[pl.ds(row0 * D, GRP * D)], in_v)
        pltpu.sync_copy(tt_hbm.at[pl.ds(row0, GRP)], tt_v)
        gb_off = tt_v[...] * (2 * D)

        UNR = 16

        def stat_body(s, carry):
            j0 = s * UNR
            accs = list(carry)
            for u in range(UNR):
                v = plsc.load_gather(in_v, [row_off + (j0 + u)])
                k = u % 4
                accs[k] = accs[k] + v
                accs[4 + k] = accs[4 + k] + v * v
            return tuple(accs)

        z = jnp.zeros((16,), jnp.float32)
        accs = lax.fori_loop(0, D // UNR, stat_body, (z,) * 8)
        acc = accs[0] + accs[1] + accs[2] + accs[3]
        acc2 = accs[4] + accs[5] + accs[6] + accs[7]
        mean = acc * (1.0 / D)
        var = acc2 * (1.0 / D) - mean * mean
        inv = _rsqrt16(var + EPS)

        def apply_body(s, _):
            j0 = s * UNR
            for u in range(UNR):
                j = j0 + u
                v = plsc.load_gather(in_v, [row_off + j])
                gj = plsc.load_gather(gb_v, [gb_off + j])
                bj = plsc.load_gather(gb_v, [gb_off + (j + D)])
                plsc.store_scatter(out_v, [row_off + j],
                                   (v - mean) * inv * gj + bj)
            return 0

        lax.fori_loop(0, D // UNR, apply_body, 0)
        pltpu.sync_copy(out_v, o_hbm.at[pl.ds(row0 * D, GRP * D)])
        return 0

    lax.fori_loop(0, rows_per_w // GRP, group_body, 0)


def sc_kernel(xf, tt2, gbf):
    n_elts = xf.shape[0]
    mesh = plsc.VectorSubcoreMesh(core_axis_name="c", subcore_axis_name="s")
    fn = pl.kernel(
        _sc_body,
        mesh=mesh,
        out_type=jax.ShapeDtypeStruct((n_elts,), jnp.float32),
        scratch_types=[
            pltpu.VMEM((GRP * D,), jnp.float32),
            pltpu.VMEM((GRP,), jnp.int32),
            pltpu.VMEM((NUM_GROUPS * 2 * D,), jnp.float32),
            pltpu.VMEM((GRP * D,), jnp.float32),
        ],
        compiler_params=pltpu.CompilerParams(
            use_tc_tiling_on_sc=False, needs_layout_passes=False),
    )
    return fn(xf, tt2, gbf)


def kernel(x, token_types, gamma, beta):
    B, S, D_ = x.shape
    n_tok = B * S
    xf = x.reshape(n_tok * D_)
    tt2 = token_types.reshape(n_tok).astype(jnp.int32)
    gbf = jnp.concatenate([gamma, beta], axis=1).reshape(-1)
    return sc_kernel(xf, tt2, gbf).reshape(B, S, D_)


# speedup vs baseline: 2.4677x; 2.4677x over previous
import functools
import jax
import jax.numpy as jnp
from jax import lax
from jax.experimental import pallas as pl
from jax.experimental.pallas import tpu as pltpu
from jax.experimental.pallas import tpu_sc as plsc

EPS = 1e-06
NUM_GROUPS = 4
D = 1024
GRP = 16


def _rsqrt16(v):
    i = plsc.bitcast(v, jnp.int32)
    i = jnp.int32(0x5F3759DF) - (i >> 1)
    y = plsc.bitcast(i, jnp.float32)
    half = v * 0.5
    for _ in range(3):
        y = y * (1.5 - half * y * y)
    return y


def _sc_body(x_hbm, tt_hbm, gb_hbm, o_hbm, in_v, tt_v, gb_v, out_v):
    n_elts = x_hbm.shape[0]
    n_tok = n_elts // D
    nw = 32
    rows_per_w = n_tok // nw
    wid = lax.axis_index("s") * 2 + lax.axis_index("c")
    base = wid * rows_per_w
    pltpu.sync_copy(gb_hbm, gb_v)
    iota16 = lax.iota(jnp.int32, 16)
    row_off = iota16 * D
    rot = iota16 * 72

    def group_body(g, _):
        row0 = base + g * GRP
        pltpu.sync_copy(x_hbm.at[pl.ds(row0 * D, GRP * D)], in_v)
        pltpu.sync_copy(tt_hbm.at[pl.ds(row0, GRP)], tt_v)
        gb_off = tt_v[...] * (2 * D)

        UNR = 16

        def stat_body(s, carry):
            j0 = s * UNR
            accs = list(carry)
            for u in range(UNR):
                jr = (rot + (j0 + u)) & (D - 1)
                v = plsc.load_gather(in_v, [row_off + jr])
                k = u % 4
                accs[k] = accs[k] + v
                accs[4 + k] = accs[4 + k] + v * v
            return tuple(accs)

        z = jnp.zeros((16,), jnp.float32)
        accs = lax.fori_loop(0, D // UNR, stat_body, (z,) * 8)
        acc = accs[0] + accs[1] + accs[2] + accs[3]
        acc2 = accs[4] + accs[5] + accs[6] + accs[7]
        mean = acc * (1.0 / D)
        var = acc2 * (1.0 / D) - mean * mean
        inv = _rsqrt16(var + EPS)

        def apply_body(s, _):
            j0 = s * UNR
            for u in range(UNR):
                jr = (rot + (j0 + u)) & (D - 1)
                v = plsc.load_gather(in_v, [row_off + jr])
                gj = plsc.load_gather(gb_v, [gb_off + jr])
                bj = plsc.load_gather(gb_v, [gb_off + (jr + D)])
                plsc.store_scatter(out_v, [row_off + jr],
                                   (v - mean) * inv * gj + bj)
            return 0

        lax.fori_loop(0, D // UNR, apply_body, 0)
        pltpu.sync_copy(out_v, o_hbm.at[pl.ds(row0 * D, GRP * D)])
        return 0

    lax.fori_loop(0, rows_per_w // GRP, group_body, 0)


def sc_kernel(xf, tt2, gbf):
    n_elts = xf.shape[0]
    mesh = plsc.VectorSubcoreMesh(core_axis_name="c", subcore_axis_name="s")
    fn = pl.kernel(
        _sc_body,
        mesh=mesh,
        out_type=jax.ShapeDtypeStruct((n_elts,), jnp.float32),
        scratch_types=[
            pltpu.VMEM((GRP * D,), jnp.float32),
            pltpu.VMEM((GRP,), jnp.int32),
            pltpu.VMEM((NUM_GROUPS * 2 * D,), jnp.float32),
            pltpu.VMEM((GRP * D,), jnp.float32),
        ],
        compiler_params=pltpu.CompilerParams(
            use_tc_tiling_on_sc=False, needs_layout_passes=False),
    )
    return fn(xf, tt2, gbf)


def kernel(x, token_types, gamma, beta):
    B, S, D_ = x.shape
    n_tok = B * S
    xf = x.reshape(n_tok * D_)
    tt2 = token_types.reshape(n_tok).astype(jnp.int32)
    gbf = jnp.concatenate([gamma, beta], axis=1).reshape(-1)
    return sc_kernel(xf, tt2, gbf).reshape(B, S, D_)


# speedup vs baseline: 2.8085x; 1.1381x over previous
import functools
import jax
import jax.numpy as jnp
from jax import lax
from jax.experimental import pallas as pl
from jax.experimental.pallas import tpu as pltpu
from jax.experimental.pallas import tpu_sc as plsc

EPS = 1e-06
NUM_GROUPS = 4
D = 1024
GRP = 16


def _rsqrt16(v):
    i = plsc.bitcast(v, jnp.int32)
    i = jnp.int32(0x5F3759DF) - (i >> 1)
    y = plsc.bitcast(i, jnp.float32)
    half = v * 0.5
    for _ in range(3):
        y = y * (1.5 - half * y * y)
    return y


def _sc_body(x_hbm, tt_hbm, gb_hbm, o_hbm, in_v, tt_v, gb_v, out_v):
    n_elts = x_hbm.shape[0]
    n_tok = n_elts // D
    nw = 32
    rows_per_w = n_tok // nw
    wid = lax.axis_index("s") * 2 + lax.axis_index("c")
    base = wid * rows_per_w
    pltpu.sync_copy(gb_hbm, gb_v)
    iota16 = lax.iota(jnp.int32, 16)
    row_off = iota16 * D
    rot = iota16 * 65

    def group_body(g, _):
        row0 = base + g * GRP
        pltpu.sync_copy(x_hbm.at[pl.ds(row0 * D, GRP * D)], in_v)
        pltpu.sync_copy(tt_hbm.at[pl.ds(row0, GRP)], tt_v)
        gb_off = tt_v[...] * (2 * D)

        UNR = 16

        def stat_body(s, carry):
            j0 = s * UNR
            accs = list(carry)
            for u in range(UNR):
                jr = (rot + (j0 + u)) & (D - 1)
                v = plsc.load_gather(in_v, [row_off + jr])
                k = u % 4
                accs[k] = accs[k] + v
                accs[4 + k] = accs[4 + k] + v * v
            return tuple(accs)

        z = jnp.zeros((16,), jnp.float32)
        accs = lax.fori_loop(0, D // UNR, stat_body, (z,) * 8)
        acc = accs[0] + accs[1] + accs[2] + accs[3]
        acc2 = accs[4] + accs[5] + accs[6] + accs[7]
        mean = acc * (1.0 / D)
        var = acc2 * (1.0 / D) - mean * mean
        inv = _rsqrt16(var + EPS)

        def apply_body(s, _):
            j0 = s * UNR
            for u in range(UNR):
                jr = (rot + (j0 + u)) & (D - 1)
                v = plsc.load_gather(in_v, [row_off + jr])
                gj = plsc.load_gather(gb_v, [gb_off + jr])
                bj = plsc.load_gather(gb_v, [gb_off + (jr + D)])
                plsc.store_scatter(out_v, [row_off + jr],
                                   (v - mean) * inv * gj + bj)
            return 0

        lax.fori_loop(0, D // UNR, apply_body, 0)
        pltpu.sync_copy(out_v, o_hbm.at[pl.ds(row0 * D, GRP * D)])
        return 0

    lax.fori_loop(0, rows_per_w // GRP, group_body, 0)


def sc_kernel(xf, tt2, gbf):
    n_elts = xf.shape[0]
    mesh = plsc.VectorSubcoreMesh(core_axis_name="c", subcore_axis_name="s")
    fn = pl.kernel(
        _sc_body,
        mesh=mesh,
        out_type=jax.ShapeDtypeStruct((n_elts,), jnp.float32),
        scratch_types=[
            pltpu.VMEM((GRP * D,), jnp.float32),
            pltpu.VMEM((GRP,), jnp.int32),
            pltpu.VMEM((NUM_GROUPS * 2 * D,), jnp.float32),
            pltpu.VMEM((GRP * D,), jnp.float32),
        ],
        compiler_params=pltpu.CompilerParams(
            use_tc_tiling_on_sc=False, needs_layout_passes=False),
    )
    return fn(xf, tt2, gbf)


def kernel(x, token_types, gamma, beta):
    B, S, D_ = x.shape
    n_tok = B * S
    xf = x.reshape(n_tok * D_)
    tt2 = token_types.reshape(n_tok).astype(jnp.int32)
    gbf = jnp.concatenate([gamma, beta], axis=1).reshape(-1)
    return sc_kernel(xf, tt2, gbf).reshape(B, S, D_)


# speedup vs baseline: 26.8170x; 9.5485x over previous
import jax
import jax.numpy as jnp
from jax.experimental import pallas as pl

EPS = 1e-06
NUM_GROUPS = 4
BLOCK_T = 1024


def _glnorm_kernel(x_ref, tt_ref, g_ref, b_ref, o_ref):
    x = x_ref[...]
    tt = tt_ref[...]
    d = x.shape[1]
    mean = jnp.mean(x, axis=1, keepdims=True)
    xc = x - mean
    var = jnp.mean(xc * xc, axis=1, keepdims=True)
    inv = jax.lax.rsqrt(var + EPS)
    onehot = (tt == jnp.arange(NUM_GROUPS)[None, :]).astype(jnp.float32)
    gg = jax.lax.dot(onehot, g_ref[...])
    bb = jax.lax.dot(onehot, b_ref[...])
    o_ref[...] = xc * (inv * gg) + bb


def kernel(x, token_types, gamma, beta):
    B, S, D = x.shape
    n_tok = B * S
    x2 = x.reshape(n_tok, D)
    tt2 = token_types.reshape(n_tok, 1).astype(jnp.int32)
    grid = (n_tok // BLOCK_T,)
    out = pl.pallas_call(
        _glnorm_kernel,
        grid=grid,
        in_specs=[
            pl.BlockSpec((BLOCK_T, D), lambda i: (i, 0)),
            pl.BlockSpec((BLOCK_T, 1), lambda i: (i, 0)),
            pl.BlockSpec((NUM_GROUPS, D), lambda i: (0, 0)),
            pl.BlockSpec((NUM_GROUPS, D), lambda i: (0, 0)),
        ],
        out_specs=pl.BlockSpec((BLOCK_T, D), lambda i: (i, 0)),
        out_shape=jax.ShapeDtypeStruct((n_tok, D), x.dtype),
    )(x2, tt2, gamma, beta)
    return out.reshape(B, S, D)
